# baseline (speedup 1.0000x reference)
import dataclasses
import functools

import jax
import jax.numpy as jnp
from jax import lax
from jax.experimental import pallas as pl
from jax.experimental.pallas import tpu as pltpu
from jax.experimental.pallas import tpu_sc as plsc

_NC = 2
_NS = 16
_L = 16


def _argmax_body(s_ref, out_ref):
    v = s_ref[...]
    m = jnp.max(v, axis=1, keepdims=True)
    io = lax.broadcasted_iota(jnp.int32, v.shape, 1)
    idx = jnp.min(jnp.where(v == m, io, v.shape[1]), axis=1)
    out_ref[0, 0, :] = idx


def _rows_argmax(s2, rows_per_block):
    nrows, e = s2.shape
    nblk = nrows // rows_per_block
    out = pl.pallas_call(
        _argmax_body,
        grid=(nblk,),
        in_specs=[pl.BlockSpec((rows_per_block, e), lambda i: (i, 0))],
        out_specs=pl.BlockSpec((1, 1, rows_per_block), lambda i: (i, 0, 0)),
        out_shape=jax.ShapeDtypeStruct((nblk, 1, rows_per_block), jnp.int32),
    )(s2)
    return out.reshape(nrows)


def _sc_table_gather(table_flat, pair_idx, a_i32, env_size):
    batch = a_i32.shape[0]
    nw = _NC * _NS
    bpw = batch // nw
    nrow = bpw // 128
    nch = bpw // _L
    mesh = plsc.VectorSubcoreMesh(core_axis_name="c", subcore_axis_name="s")
    cp = pltpu.CompilerParams()
    if "needs_layout_passes" in pltpu.CompilerParams.__dataclass_fields__:
        cp = dataclasses.replace(cp, needs_layout_passes=False)

    @functools.partial(
        pl.kernel,
        compiler_params=cp,
        out_type=jax.ShapeDtypeStruct((batch // 128, 128), jnp.float32),
        mesh=mesh,
        scratch_types=[
            pltpu.VMEM((2 * bpw,), jnp.int32),
            pltpu.VMEM((bpw,), jnp.int32),
            pltpu.VMEM((nrow, 128), jnp.int32),
            pltpu.VMEM((nrow, 128), jnp.float32),
            pltpu.SemaphoreType.DMA,
        ],
    )
    def k(table_hbm, pair_hbm, a_hbm, out_hbm, pair_v, a_v, flat_v, val_v,
          sem):
        wid = lax.axis_index("s") * _NC + lax.axis_index("c")
        base = wid * bpw
        pltpu.sync_copy(pair_hbm.at[pl.ds(2 * base, 2 * bpw)], pair_v)
        pltpu.sync_copy(a_hbm.at[pl.ds(base, bpw)], a_v)

        lane = lax.iota(jnp.int32, _L)

        @pl.loop(0, nch)
        def _(c):
            off = c * _L
            ev = 2 * off + 2 * lane
            xi = plsc.load_gather(pair_v, [ev])
            yi = plsc.load_gather(pair_v, [ev + 1])
            av = a_v[pl.ds(off, _L)]
            flat = xi * (4 * env_size) + yi * 4 + av
            row = c // 8
            col = (c % 8) * _L
            flat_v.at[row][pl.ds(col, _L)] = flat

        copies = [
            pltpu.async_copy(table_hbm.at[flat_v.at[j]], val_v.at[j], sem)
            for j in range(nrow)
        ]
        for cp in copies:
            cp.wait()
        pltpu.sync_copy(val_v, out_hbm.at[pl.ds(wid * nrow, nrow)])

    return k(table_flat, pair_idx, a_i32).reshape(batch)


def kernel(s, a, env_size, table):
    batch = s.shape[0]
    e = s.shape[2]
    s2 = s.reshape(2 * batch, e)
    pair_idx = _rows_argmax(s2, rows_per_block=2048)
    return _sc_table_gather(table.reshape(-1), pair_idx,
                            a.astype(jnp.int32), e)


# speedup vs baseline: 16.4150x; 16.4150x over previous
import dataclasses
import functools

import jax
import jax.numpy as jnp
from jax import lax
from jax.experimental import pallas as pl
from jax.experimental.pallas import tpu as pltpu
from jax.experimental.pallas import tpu_sc as plsc

_NC = 2
_NS = 16


def _argmax_body(s_ref, a_ref, out_ref):
    v = s_ref[...]
    e = v.shape[1]
    m = jnp.max(v, axis=1, keepdims=True)
    io = lax.broadcasted_iota(jnp.int32, v.shape, 1)
    idx = jnp.min(jnp.where(v == m, io, e), axis=1)
    x = idx[0]
    y = idx[1]
    av = a_ref[0, 0, :]
    out_ref[0, 0, :] = (x << 12) + (av << 10) + y


def _tc_argmax_phys(st, a3, bb):
    two, e, batch = st.shape
    nblk = batch // bb
    return pl.pallas_call(
        _argmax_body,
        grid=(nblk,),
        in_specs=[
            pl.BlockSpec((2, e, bb), lambda i: (0, 0, i)),
            pl.BlockSpec((1, 1, bb), lambda i: (i, 0, 0)),
        ],
        out_specs=pl.BlockSpec((1, 1, bb), lambda i: (i, 0, 0)),
        out_shape=jax.ShapeDtypeStruct((nblk, 1, bb), jnp.int32),
    )(st, a3)


def _sc_cp():
    cp = pltpu.CompilerParams()
    if "needs_layout_passes" in pltpu.CompilerParams.__dataclass_fields__:
        cp = dataclasses.replace(cp, needs_layout_passes=False)
    return cp


def _sc_table_repack(t2):
    nw = _NC * _NS
    ex, na, ey = t2.shape
    mesh = plsc.VectorSubcoreMesh(core_axis_name="c", subcore_axis_name="s")
    starts = list(range(0, ey - 16, 16)) + [ey - 16]

    @functools.partial(
        pl.kernel,
        out_type=jax.ShapeDtypeStruct((ex * 4096,), jnp.float32),
        mesh=mesh,
        compiler_params=_sc_cp(),
        scratch_types=[
            pltpu.VMEM((na, ey), jnp.float32),
            pltpu.VMEM((4096,), jnp.float32),
            pltpu.SemaphoreType.DMA,
        ],
    )
    def k(t_hbm, o_hbm, buf, buf2, sem):
        wid = lax.axis_index("s") * _NC + lax.axis_index("c")

        @pl.loop(0, (ex + nw - 1) // nw)
        def _(g):
            x = wid + g * nw

            @pl.when(x < ex)
            def _():
                pltpu.async_copy(t_hbm.at[x], buf, sem).wait()
                for j in range(na):
                    @pl.loop(0, len(starts))
                    def _(ci, j=j):
                        c = _chunk_start(ci, ey)
                        buf2[pl.ds(j * 1024 + c, 16)] = (
                            buf[j, pl.ds(c, 16)])
                pltpu.async_copy(
                    buf2, o_hbm.at[pl.ds(x * 4096, 4096)], sem).wait()

    return k(t2)


def _chunk_start(ci, ey):
    return jnp.minimum(16 * ci, ey - 16)


def _sc_table_gather(tlin, p2):
    nrows = p2.shape[0]
    nw = _NC * _NS
    rpw = nrows // nw
    mesh = plsc.VectorSubcoreMesh(core_axis_name="c", subcore_axis_name="s")

    @functools.partial(
        pl.kernel,
        out_type=jax.ShapeDtypeStruct((nrows, 128), jnp.float32),
        mesh=mesh,
        compiler_params=_sc_cp(),
        scratch_types=[
            pltpu.VMEM((rpw, 128), jnp.int32),
            pltpu.VMEM((rpw, 128), jnp.float32),
            pltpu.SemaphoreType.DMA,
        ],
    )
    def k(t_hbm, p_hbm, o_hbm, iv, vv, sem):
        wid = lax.axis_index("s") * _NC + lax.axis_index("c")
        pltpu.sync_copy(p_hbm.at[pl.ds(rpw * wid, rpw)], iv)
        copies = [
            pltpu.async_copy(t_hbm.at[iv.at[j]], vv.at[j], sem)
            for j in range(rpw)
        ]
        for c in copies:
            c.wait()
        pltpu.sync_copy(vv, o_hbm.at[pl.ds(rpw * wid, rpw)])

    return k(tlin, p2)


def kernel(s, a, env_size, table):
    batch = s.shape[0]
    st = jnp.transpose(s, (1, 2, 0))
    a3 = a.astype(jnp.int32).reshape(-1, 1, 2048)
    p = _tc_argmax_phys(st, a3, bb=2048)
    p2 = p.reshape(batch // 128, 128)
    t2 = jnp.transpose(table, (0, 2, 1))
    tlin = _sc_table_repack(t2)
    out2 = _sc_table_gather(tlin, p2)
    return out2.reshape(batch)


# speedup vs baseline: 22.6911x; 1.3823x over previous
import dataclasses
import functools

import jax
import jax.numpy as jnp
from jax import lax
from jax.experimental import pallas as pl
from jax.experimental.pallas import tpu as pltpu
from jax.experimental.pallas import tpu_sc as plsc

_NC = 2
_NS = 16


def _argmax_body(s_ref, a_ref, out_ref):
    v = s_ref[...]
    e = v.shape[1]
    m = jnp.max(v, axis=1, keepdims=True)
    io = lax.broadcasted_iota(jnp.int32, v.shape, 1)
    idx = jnp.min(jnp.where(v == m, io, e), axis=1)
    x = idx[0]
    y = idx[1]
    av = a_ref[0, 0, :]
    out_ref[0, 0, :] = (x << 12) + (av << 10) + y


def _tc_argmax_phys(st, a3, bb):
    two, e, batch = st.shape
    nblk = batch // bb
    return pl.pallas_call(
        _argmax_body,
        grid=(nblk,),
        in_specs=[
            pl.BlockSpec((2, e, bb), lambda i: (0, 0, i)),
            pl.BlockSpec((1, 1, bb), lambda i: (i, 0, 0)),
        ],
        out_specs=pl.BlockSpec((1, 1, bb), lambda i: (i, 0, 0)),
        out_shape=jax.ShapeDtypeStruct((nblk, 1, bb), jnp.int32),
    )(st, a3)


def _sc_cp():
    cp = pltpu.CompilerParams()
    if "needs_layout_passes" in pltpu.CompilerParams.__dataclass_fields__:
        cp = dataclasses.replace(cp, needs_layout_passes=False)
    return cp


def _sc_table_repack(t2):
    nw = _NC * _NS
    ex, na, ey = t2.shape
    mesh = plsc.VectorSubcoreMesh(core_axis_name="c", subcore_axis_name="s")
    starts = list(range(0, ey - 16, 16)) + [ey - 16]

    ng = (ex + nw - 1) // nw

    @functools.partial(
        pl.kernel,
        out_type=jax.ShapeDtypeStruct((ex * 4096,), jnp.float32),
        mesh=mesh,
        compiler_params=_sc_cp(),
        scratch_types=[
            pltpu.VMEM((na, ey), jnp.float32),
            pltpu.VMEM((na, ey), jnp.float32),
            pltpu.VMEM((4096,), jnp.float32),
            pltpu.VMEM((4096,), jnp.float32),
            pltpu.SemaphoreType.DMA,
            pltpu.SemaphoreType.DMA,
        ],
    )
    def k(t_hbm, o_hbm, buf_a, buf_b, out_a, out_b, semi, semo):
        wid = lax.axis_index("s") * _NC + lax.axis_index("c")

        def fire_in(g, buf):
            x = wid + nw * g

            @pl.when(x < ex)
            def _():
                pltpu.async_copy(t_hbm.at[x], buf, semi)

        def step(g, buf, ob):
            x = wid + nw * g

            @pl.when(x < ex)
            def _():
                @pl.when(g >= 2)
                def _():
                    pltpu.make_async_copy(
                        ob, o_hbm.at[pl.ds(0, 4096)], semo).wait()
                pltpu.make_async_copy(t_hbm.at[x], buf, semi).wait()
                for j in range(na):
                    for c in starts:
                        ob[pl.ds(j * 1024 + c, 16)] = buf[j, pl.ds(c, 16)]
                pltpu.async_copy(
                    ob, o_hbm.at[pl.ds(x * 4096, 4096)], semo)
                fire_in(g + 2, buf)

        fire_in(0, buf_a)
        fire_in(1, buf_b)

        @pl.loop(0, ng // 2)
        def _(i):
            step(2 * i, buf_a, out_a)
            step(2 * i + 1, buf_b, out_b)

        pltpu.make_async_copy(out_a, o_hbm.at[pl.ds(0, 4096)], semo).wait()
        pltpu.make_async_copy(out_b, o_hbm.at[pl.ds(0, 4096)], semo).wait()

    return k(t2)


def _sc_table_gather(tlin, p2):
    nrows = p2.shape[0]
    nw = _NC * _NS
    rpw = nrows // nw
    mesh = plsc.VectorSubcoreMesh(core_axis_name="c", subcore_axis_name="s")

    @functools.partial(
        pl.kernel,
        out_type=jax.ShapeDtypeStruct((nrows, 128), jnp.float32),
        mesh=mesh,
        compiler_params=_sc_cp(),
        scratch_types=[
            pltpu.VMEM((rpw, 128), jnp.int32),
            pltpu.VMEM((rpw, 128), jnp.float32),
            pltpu.SemaphoreType.DMA,
        ],
    )
    def k(t_hbm, p_hbm, o_hbm, iv, vv, sem):
        wid = lax.axis_index("s") * _NC + lax.axis_index("c")
        pltpu.sync_copy(p_hbm.at[pl.ds(rpw * wid, rpw)], iv)
        copies = [
            pltpu.async_copy(t_hbm.at[iv.at[j]], vv.at[j], sem)
            for j in range(rpw)
        ]
        for c in copies:
            c.wait()
        pltpu.sync_copy(vv, o_hbm.at[pl.ds(rpw * wid, rpw)])

    return k(tlin, p2)


def kernel(s, a, env_size, table):
    batch = s.shape[0]
    st = jnp.transpose(s, (1, 2, 0))
    a3 = a.astype(jnp.int32).reshape(-1, 1, 2048)
    p = _tc_argmax_phys(st, a3, bb=2048)
    p2 = p.reshape(batch // 128, 128)
    t2 = jnp.transpose(table, (0, 2, 1))
    tlin = _sc_table_repack(t2)
    out2 = _sc_table_gather(tlin, p2)
    return out2.reshape(batch)


# speedup vs baseline: 22.6927x; 1.0001x over previous
import dataclasses
import functools

import jax
import jax.numpy as jnp
from jax import lax
from jax.experimental import pallas as pl
from jax.experimental.pallas import tpu as pltpu
from jax.experimental.pallas import tpu_sc as plsc

_NC = 2
_NS = 16


def _argmax_body(s_ref, a_ref, out_ref):
    v = s_ref[...]
    e = v.shape[1]
    m = jnp.max(v, axis=1, keepdims=True)
    io = lax.broadcasted_iota(jnp.int32, v.shape, 1)
    idx = jnp.min(jnp.where(v == m, io, e), axis=1)
    x = idx[0]
    y = idx[1]
    av = a_ref[0, 0, :]
    out_ref[0, 0, :] = (x << 12) + (av << 10) + y


def _tc_argmax_phys(st, a3, bb):
    two, e, batch = st.shape
    nblk = batch // bb
    return pl.pallas_call(
        _argmax_body,
        grid=(nblk,),
        in_specs=[
            pl.BlockSpec((2, e, bb), lambda i: (0, 0, i)),
            pl.BlockSpec((1, 1, bb), lambda i: (i, 0, 0)),
        ],
        out_specs=pl.BlockSpec((1, 1, bb), lambda i: (i, 0, 0)),
        out_shape=jax.ShapeDtypeStruct((nblk, 1, bb), jnp.int32),
        compiler_params=pltpu.CompilerParams(skip_device_barrier=True),
    )(st, a3)


def _sc_cp():
    cp = pltpu.CompilerParams(skip_device_barrier=True)
    if "needs_layout_passes" in pltpu.CompilerParams.__dataclass_fields__:
        cp = dataclasses.replace(cp, needs_layout_passes=False)
    return cp


def _sc_table_repack(t2):
    nw = _NC * _NS
    ex, na, ey = t2.shape
    mesh = plsc.VectorSubcoreMesh(core_axis_name="c", subcore_axis_name="s")
    starts = list(range(0, ey - 16, 16)) + [ey - 16]

    ng = (ex + nw - 1) // nw

    @functools.partial(
        pl.kernel,
        out_type=jax.ShapeDtypeStruct((ex * 4096,), jnp.float32),
        mesh=mesh,
        compiler_params=_sc_cp(),
        scratch_types=[
            pltpu.VMEM((na, ey), jnp.float32),
            pltpu.VMEM((na, ey), jnp.float32),
            pltpu.VMEM((4096,), jnp.float32),
            pltpu.VMEM((4096,), jnp.float32),
            pltpu.SemaphoreType.DMA,
            pltpu.SemaphoreType.DMA,
        ],
    )
    def k(t_hbm, o_hbm, buf_a, buf_b, out_a, out_b, semi, semo):
        wid = lax.axis_index("s") * _NC + lax.axis_index("c")

        def fire_in(g, buf):
            x = wid + nw * g

            @pl.when(x < ex)
            def _():
                pltpu.async_copy(t_hbm.at[x], buf, semi)

        def step(g, buf, ob):
            x = wid + nw * g

            @pl.when(x < ex)
            def _():
                @pl.when(g >= 2)
                def _():
                    pltpu.make_async_copy(
                        ob, o_hbm.at[pl.ds(0, 4096)], semo).wait()
                pltpu.make_async_copy(t_hbm.at[x], buf, semi).wait()
                for j in range(na):
                    for c in starts:
                        ob[pl.ds(j * 1024 + c, 16)] = buf[j, pl.ds(c, 16)]
                pltpu.async_copy(
                    ob, o_hbm.at[pl.ds(x * 4096, 4096)], semo)
                fire_in(g + 2, buf)

        fire_in(0, buf_a)
        fire_in(1, buf_b)

        @pl.loop(0, ng // 2)
        def _(i):
            step(2 * i, buf_a, out_a)
            step(2 * i + 1, buf_b, out_b)

        pltpu.make_async_copy(out_a, o_hbm.at[pl.ds(0, 4096)], semo).wait()
        pltpu.make_async_copy(out_b, o_hbm.at[pl.ds(0, 4096)], semo).wait()

    return k(t2)


def _sc_table_gather(tlin, p2):
    nrows = p2.shape[0]
    nw = _NC * _NS
    rpw = nrows // nw
    mesh = plsc.VectorSubcoreMesh(core_axis_name="c", subcore_axis_name="s")

    @functools.partial(
        pl.kernel,
        out_type=jax.ShapeDtypeStruct((nrows, 128), jnp.float32),
        mesh=mesh,
        compiler_params=_sc_cp(),
        scratch_types=[
            pltpu.VMEM((rpw, 128), jnp.int32),
            pltpu.VMEM((rpw, 128), jnp.float32),
            pltpu.SemaphoreType.DMA,
        ],
    )
    def k(t_hbm, p_hbm, o_hbm, iv, vv, sem):
        wid = lax.axis_index("s") * _NC + lax.axis_index("c")
        pltpu.sync_copy(p_hbm.at[pl.ds(rpw * wid, rpw)], iv)
        copies = [
            pltpu.async_copy(t_hbm.at[iv.at[j]], vv.at[j], sem)
            for j in range(rpw)
        ]
        for c in copies:
            c.wait()
        pltpu.sync_copy(vv, o_hbm.at[pl.ds(rpw * wid, rpw)])

    return k(tlin, p2)


def kernel(s, a, env_size, table):
    batch = s.shape[0]
    st = jnp.transpose(s, (1, 2, 0))
    a3 = a.astype(jnp.int32).reshape(-1, 1, 2048)
    p = _tc_argmax_phys(st, a3, bb=2048)
    p2 = p.reshape(batch // 128, 128)
    t2 = jnp.transpose(table, (0, 2, 1))
    tlin = _sc_table_repack(t2)
    out2 = _sc_table_gather(tlin, p2)
    return out2.reshape(batch)


# speedup vs baseline: 22.7053x; 1.0006x over previous
import dataclasses
import functools

import jax
import jax.numpy as jnp
from jax import lax
from jax.experimental import pallas as pl
from jax.experimental.pallas import tpu as pltpu
from jax.experimental.pallas import tpu_sc as plsc

_NC = 2
_NS = 16


def _argmax_body(s_ref, a_ref, out_ref):
    v = s_ref[...]
    e = v.shape[1]
    m = jnp.max(v, axis=1, keepdims=True)
    io = lax.broadcasted_iota(jnp.int32, v.shape, 1)
    idx = jnp.min(jnp.where(v == m, io, e), axis=1)
    x = idx[0]
    y = idx[1]
    av = a_ref[0, 0, :]
    out_ref[0, 0, :] = (x << 12) + (av << 10) + y


def _tc_argmax_phys(st, a3, bb):
    two, e, batch = st.shape
    nblk = batch // bb
    return pl.pallas_call(
        _argmax_body,
        grid=(nblk,),
        in_specs=[
            pl.BlockSpec((2, e, bb), lambda i: (0, 0, i)),
            pl.BlockSpec((1, 1, bb), lambda i: (i, 0, 0)),
        ],
        out_specs=pl.BlockSpec((1, 1, bb), lambda i: (i, 0, 0)),
        out_shape=jax.ShapeDtypeStruct((nblk, 1, bb), jnp.int32),
        compiler_params=pltpu.CompilerParams(
            skip_device_barrier=True,
            dimension_semantics=("parallel",),
        ),
    )(st, a3)


def _sc_cp():
    cp = pltpu.CompilerParams(skip_device_barrier=True)
    if "needs_layout_passes" in pltpu.CompilerParams.__dataclass_fields__:
        cp = dataclasses.replace(cp, needs_layout_passes=False)
    return cp


def _sc_table_repack(t2):
    nw = _NC * _NS
    ex, na, ey = t2.shape
    mesh = plsc.VectorSubcoreMesh(core_axis_name="c", subcore_axis_name="s")
    starts = list(range(0, ey - 16, 16)) + [ey - 16]

    ng = (ex + nw - 1) // nw

    @functools.partial(
        pl.kernel,
        out_type=jax.ShapeDtypeStruct((ex * 4096,), jnp.float32),
        mesh=mesh,
        compiler_params=_sc_cp(),
        scratch_types=[
            pltpu.VMEM((na, ey), jnp.float32),
            pltpu.VMEM((na, ey), jnp.float32),
            pltpu.VMEM((4096,), jnp.float32),
            pltpu.VMEM((4096,), jnp.float32),
            pltpu.SemaphoreType.DMA,
            pltpu.SemaphoreType.DMA,
        ],
    )
    def k(t_hbm, o_hbm, buf_a, buf_b, out_a, out_b, semi, semo):
        wid = lax.axis_index("s") * _NC + lax.axis_index("c")

        def fire_in(g, buf):
            x = wid + nw * g

            @pl.when(x < ex)
            def _():
                pltpu.async_copy(t_hbm.at[x], buf, semi)

        def step(g, buf, ob):
            x = wid + nw * g

            @pl.when(x < ex)
            def _():
                @pl.when(g >= 2)
                def _():
                    pltpu.make_async_copy(
                        ob, o_hbm.at[pl.ds(0, 4096)], semo).wait()
                pltpu.make_async_copy(t_hbm.at[x], buf, semi).wait()
                for j in range(na):
                    for c in starts:
                        ob[pl.ds(j * 1024 + c, 16)] = buf[j, pl.ds(c, 16)]
                pltpu.async_copy(
                    ob, o_hbm.at[pl.ds(x * 4096, 4096)], semo)
                fire_in(g + 2, buf)

        fire_in(0, buf_a)
        fire_in(1, buf_b)

        @pl.loop(0, ng // 2)
        def _(i):
            step(2 * i, buf_a, out_a)
            step(2 * i + 1, buf_b, out_b)

        pltpu.make_async_copy(out_a, o_hbm.at[pl.ds(0, 4096)], semo).wait()
        pltpu.make_async_copy(out_b, o_hbm.at[pl.ds(0, 4096)], semo).wait()

    return k(t2)


def _sc_table_gather(tlin, p2):
    nrows = p2.shape[0]
    nw = _NC * _NS
    rpw = nrows // nw
    mesh = plsc.VectorSubcoreMesh(core_axis_name="c", subcore_axis_name="s")

    @functools.partial(
        pl.kernel,
        out_type=jax.ShapeDtypeStruct((nrows, 128), jnp.float32),
        mesh=mesh,
        compiler_params=_sc_cp(),
        scratch_types=[
            pltpu.VMEM((rpw, 128), jnp.int32),
            pltpu.VMEM((rpw, 128), jnp.float32),
            pltpu.SemaphoreType.DMA,
        ],
    )
    def k(t_hbm, p_hbm, o_hbm, iv, vv, sem):
        wid = lax.axis_index("s") * _NC + lax.axis_index("c")
        pltpu.sync_copy(p_hbm.at[pl.ds(rpw * wid, rpw)], iv)
        copies = [
            pltpu.async_copy(t_hbm.at[iv.at[j]], vv.at[j], sem)
            for j in range(rpw)
        ]
        for c in copies:
            c.wait()
        pltpu.sync_copy(vv, o_hbm.at[pl.ds(rpw * wid, rpw)])

    return k(tlin, p2)


def kernel(s, a, env_size, table):
    batch = s.shape[0]
    st = jnp.transpose(s, (1, 2, 0))
    a3 = a.astype(jnp.int32).reshape(-1, 1, 2048)
    p = _tc_argmax_phys(st, a3, bb=2048)
    p2 = p.reshape(batch // 128, 128)
    t2 = jnp.transpose(table, (0, 2, 1))
    tlin = _sc_table_repack(t2)
    out2 = _sc_table_gather(tlin, p2)
    return out2.reshape(batch)
